# Initial kernel scaffold; baseline (speedup 1.0000x reference)
import jax
import jax.numpy as jnp
from jax.experimental import pallas as pl


def kernel(x, table, W_ih, W_hh, b_ih, b_hh):
    raise NotImplementedError("write your pallas kernel here")


# speedup vs baseline: 15.4478x; 15.4478x over previous
import jax
import jax.numpy as jnp
from jax.experimental import pallas as pl
from jax.experimental.pallas import tpu as pltpu
from jax.experimental.pallas import tpu_sc as plsc

VOCAB = 100000
EMB = 128
HID = 128
B = 1024
T = 200
G = 3 * HID


_GATHER_WINDOW = 128


def _sc_gather(table, idx):
    n = idx.shape[1]
    mesh = plsc.VectorSubcoreMesh(core_axis_name="c", subcore_axis_name="s")

    def run(table, idx):
        @pl.kernel(
            out_type=jax.ShapeDtypeStruct((n, EMB), table.dtype),
            mesh=mesh,
        )
        def k(tbl_hbm, idx_hbm, out_hbm):
            def body(i_vmem, o_vmem):
                pltpu.sync_copy(tbl_hbm.at[i_vmem.at[0]], o_vmem)

            pltpu.emit_pipeline(
                body,
                grid=(n // _GATHER_WINDOW,),
                in_specs=[
                    pl.BlockSpec((1, _GATHER_WINDOW), lambda i: (0, i)),
                ],
                out_specs=[
                    pl.BlockSpec((_GATHER_WINDOW, EMB), lambda i: (i, 0)),
                ],
                core_axis_name=("c", "s"),
                dimension_semantics=(pltpu.PARALLEL,),
            )(idx_hbm, out_hbm)

        return k(table, idx)

    return run(table, idx)


_TS = 8
_NB = 2
_BB = B // _NB


def _gru_body(emb_ref, wih_ref, whh_ref, bih_ref, bhh_ref, out_ref, h_ref):
    t = pl.program_id(1)

    @pl.when(t == 0)
    def _init():
        h_ref[...] = jnp.zeros_like(h_ref)

    h = h_ref[...]
    wih = wih_ref[...]
    whh = whh_ref[...]
    bih = bih_ref[...]
    bhh = bhh_ref[...]
    for s in range(_TS):
        e = emb_ref[s]
        gi = jnp.dot(e, wih, preferred_element_type=jnp.float32) + bih
        gh = jnp.dot(h, whh, preferred_element_type=jnp.float32) + bhh
        r = jax.nn.sigmoid(gi[:, :HID] + gh[:, :HID])
        z = jax.nn.sigmoid(gi[:, HID:2 * HID] + gh[:, HID:2 * HID])
        nn = jnp.tanh(gi[:, 2 * HID:] + r * gh[:, 2 * HID:])
        h = (1.0 - z) * nn + z * h
        out_ref[:, s, :] = h
    h_ref[...] = h


def _tc_gru(emb, w_ih, w_hh, b_ih, b_hh):
    return pl.pallas_call(
        _gru_body,
        grid=(_NB, T // _TS),
        in_specs=[
            pl.BlockSpec((_TS, _BB, EMB), lambda j, t: (t, j, 0)),
            pl.BlockSpec((EMB, G), lambda j, t: (0, 0)),
            pl.BlockSpec((HID, G), lambda j, t: (0, 0)),
            pl.BlockSpec((1, G), lambda j, t: (0, 0)),
            pl.BlockSpec((1, G), lambda j, t: (0, 0)),
        ],
        out_specs=pl.BlockSpec((_BB, _TS, HID), lambda j, t: (j, t, 0)),
        out_shape=jax.ShapeDtypeStruct((B, T, HID), jnp.float32),
        scratch_shapes=[pltpu.VMEM((_BB, HID), jnp.float32)],
        compiler_params=pltpu.CompilerParams(
            dimension_semantics=("parallel", "arbitrary"),
        ),
    )(emb, w_ih, w_hh, b_ih.reshape(1, G), b_hh.reshape(1, G))


def kernel(x, table, W_ih, W_hh, b_ih, b_hh):
    idx = x.astype(jnp.int32).T.reshape(1, T * B)
    emb = _sc_gather(table, idx).reshape(T, B, EMB)
    return _tc_gru(emb, W_ih, W_hh, b_ih, b_hh)
